# baseline (speedup 1.0000x reference)
import functools

import jax
import jax.numpy as jnp
from jax import lax
from jax.experimental import pallas as pl
from jax.experimental.pallas import tpu as pltpu
from jax.experimental.pallas import tpu_sc as plsc

N = 10000
E = 320000
D = 128

NC = 2
NS = 16
NW = NC * NS
CHUNK = 128
CHUNKS = 82
NBUF = 2
EPW = CHUNKS * CHUNK
E_PAD = NW * EPW
N_PAD = 10112
ROWS_PER_SUB = N_PAD // NS


def _sc_scatter_body(h_hbm, src_hbm, dst_hbm, zeros_hbm, out_hbm,
                     src_v0, src_v1, dst_v0, dst_v1, rows_v0, rows_v1,
                     agg, gsem):
    c = lax.axis_index("c")
    s = lax.axis_index("s")
    wid = s * NC + c

    row0 = s * ROWS_PER_SUB
    pltpu.sync_copy(zeros_hbm, agg.at[pl.ds(row0, ROWS_PER_SUB)])
    plsc.subcore_barrier()

    ebase = wid * EPW

    def body(g, carry):
        off = ebase + g * (2 * CHUNK)
        pltpu.sync_copy(src_hbm.at[pl.ds(off, CHUNK)], src_v0)
        pltpu.sync_copy(src_hbm.at[pl.ds(off + CHUNK, CHUNK)], src_v1)
        pltpu.sync_copy(dst_hbm.at[pl.ds(off, CHUNK)], dst_v0)
        pltpu.sync_copy(dst_hbm.at[pl.ds(off + CHUNK, CHUNK)], dst_v1)
        pltpu.async_copy(h_hbm.at[src_v0], rows_v0, gsem)
        pltpu.async_copy(h_hbm.at[src_v1], rows_v1, gsem)
        pltpu.make_async_copy(h_hbm.at[src_v0], rows_v0, gsem).wait()
        pltpu.make_async_copy(h_hbm.at[src_v1], rows_v1, gsem).wait()
        pltpu.sync_copy(rows_v0, agg.at[dst_v0], add=True)
        pltpu.sync_copy(rows_v1, agg.at[dst_v1], add=True)
        return carry

    lax.fori_loop(0, CHUNKS // NBUF, body, 0)
    plsc.subcore_barrier()

    pltpu.sync_copy(agg.at[pl.ds(row0, ROWS_PER_SUB)],
                    out_hbm.at[c, pl.ds(row0, ROWS_PER_SUB)])


@jax.jit
def _sc_scatter(h, src_pad, dst_pad, zeros_blk):
    mesh = plsc.VectorSubcoreMesh(core_axis_name="c", subcore_axis_name="s")
    f = pl.kernel(
        _sc_scatter_body,
        out_type=jax.ShapeDtypeStruct((NC, N_PAD, D), jnp.float32),
        mesh=mesh,
        scratch_types=[
            pltpu.VMEM((CHUNK,), jnp.int32),
            pltpu.VMEM((CHUNK,), jnp.int32),
            pltpu.VMEM((CHUNK,), jnp.int32),
            pltpu.VMEM((CHUNK,), jnp.int32),
            pltpu.VMEM((CHUNK, D), jnp.float32),
            pltpu.VMEM((CHUNK, D), jnp.float32),
            pltpu.VMEM_SHARED((N_PAD, D), jnp.float32),
            pltpu.SemaphoreType.DMA,
        ],
    )
    return f(h, src_pad, dst_pad, zeros_blk)


_EPS = 1e-5
_PREC = jax.lax.Precision.HIGHEST


def _bn_relu(y, g, b):
    m = jnp.mean(y, axis=0, keepdims=True)
    v = jnp.mean((y - m) ** 2, axis=0, keepdims=True)
    return jnp.maximum(g * (y - m) * lax.rsqrt(v + _EPS) + b, 0.0)


def _mlp(h, w1, b1, g1, be1, w2, b2, g2, be2):
    y = jnp.dot(h, w1, preferred_element_type=jnp.float32, precision=_PREC) + b1
    y = _bn_relu(y, g1, be1)
    y = jnp.dot(y, w2, preferred_element_type=jnp.float32, precision=_PREC) + b2
    return _bn_relu(y, g2, be2)


def _mlp1_body(x_ref, p0_ref, p1_ref,
               w1_ref, b1_ref, g1_ref, be1_ref,
               w2_ref, b2_ref, g2_ref, be2_ref, out_ref):
    h = x_ref[...] + p0_ref[:N] + p1_ref[:N]
    out_ref[...] = _mlp(h, w1_ref[...], b1_ref[...], g1_ref[...], be1_ref[...],
                        w2_ref[...], b2_ref[...], g2_ref[...], be2_ref[...])


def _mlp2_body(h1_ref, p0_ref, p1_ref,
               w1_ref, b1_ref, g1_ref, be1_ref,
               w2_ref, b2_ref, g2_ref, be2_ref,
               wa_ref, wb_ref, lb_ref, out_ref):
    h1 = h1_ref[...]
    h = h1 + p0_ref[:N] + p1_ref[:N]
    h2 = _mlp(h, w1_ref[...], b1_ref[...], g1_ref[...], be1_ref[...],
              w2_ref[...], b2_ref[...], g2_ref[...], be2_ref[...])
    out_ref[...] = (
        jnp.dot(h1, wa_ref[...], preferred_element_type=jnp.float32,
                precision=_PREC)
        + jnp.dot(h2, wb_ref[...], preferred_element_type=jnp.float32,
                  precision=_PREC)
        + lb_ref[...])


def _vmem_specs(n):
    return [pl.BlockSpec(memory_space=pltpu.VMEM) for _ in range(n)]


def _mlp1(x, parts, p):
    return pl.pallas_call(
        _mlp1_body,
        out_shape=jax.ShapeDtypeStruct((N, D), jnp.float32),
        in_specs=_vmem_specs(11),
        out_specs=pl.BlockSpec(memory_space=pltpu.VMEM),
    )(x, parts[0], parts[1],
      p["w1"], p["b1"].reshape(1, D), p["g1"].reshape(1, D),
      p["be1"].reshape(1, D),
      p["w2"], p["b2"].reshape(1, D), p["g2"].reshape(1, D),
      p["be2"].reshape(1, D))


def _mlp2(h1, parts, p, lin_w, lin_b):
    return pl.pallas_call(
        _mlp2_body,
        out_shape=jax.ShapeDtypeStruct((N, D), jnp.float32),
        in_specs=_vmem_specs(14),
        out_specs=pl.BlockSpec(memory_space=pltpu.VMEM),
    )(h1, parts[0], parts[1],
      p["w1"], p["b1"].reshape(1, D), p["g1"].reshape(1, D),
      p["be1"].reshape(1, D),
      p["w2"], p["b2"].reshape(1, D), p["g2"].reshape(1, D),
      p["be2"].reshape(1, D),
      lin_w[:D], lin_w[D:], lin_b.reshape(1, D))


def kernel(x, edge_index, params):
    src = edge_index[0]
    dst = edge_index[1]
    pad = E_PAD - E
    src_pad = jnp.concatenate([src, jnp.zeros((pad,), jnp.int32)])
    dst_pad = jnp.concatenate([dst, jnp.full((pad,), N, jnp.int32)])
    zeros_blk = jnp.zeros((ROWS_PER_SUB, D), jnp.float32)

    parts1 = _sc_scatter(x, src_pad, dst_pad, zeros_blk)
    h1 = _mlp1(x, parts1, params["conv1"])
    parts2 = _sc_scatter(h1, src_pad, dst_pad, zeros_blk)
    return _mlp2(h1, parts2, params["conv2"], params["lin_w"], params["lin_b"])


# speedup vs baseline: 2.0849x; 2.0849x over previous
import functools

import jax
import jax.numpy as jnp
from jax import lax
from jax.experimental import pallas as pl
from jax.experimental.pallas import tpu as pltpu
from jax.experimental.pallas import tpu_sc as plsc

N = 10000
E = 320000
D = 128

NC = 2
NS = 16
NW = NC * NS
CHUNK = 128
CHUNKS = 79
EPW = CHUNKS * CHUNK
E_PAD = NW * EPW
N_PAD = 10112
ROWS_PER_SUB = N_PAD // NS


def _sc_scatter_body(h_hbm, src_hbm, dst_hbm, zeros_hbm, out_hbm,
                     src_v0, src_v1, dst_v0, dst_v1, rows_v0, rows_v1,
                     agg, gsem, ssem0, ssem1):
    c = lax.axis_index("c")
    s = lax.axis_index("s")
    wid = s * NC + c

    row0 = s * ROWS_PER_SUB
    pltpu.sync_copy(zeros_hbm, agg.at[pl.ds(row0, ROWS_PER_SUB)])
    plsc.subcore_barrier()

    ebase = wid * EPW
    bufs = ((src_v0, dst_v0, rows_v0, ssem0),
            (src_v1, dst_v1, rows_v1, ssem1))

    dummy = E_PAD - CHUNK
    for (sv, dv, rv, ssem) in bufs:
        pltpu.sync_copy(dst_hbm.at[pl.ds(dummy, CHUNK)], dv)
        pltpu.sync_copy(zeros_hbm.at[pl.ds(0, CHUNK)], rv)
        pltpu.async_copy(rv, agg.at[dv], ssem, add=True)

    def step(k, sv, dv, rv, ssem):
        off = ebase + k * CHUNK
        pltpu.sync_copy(src_hbm.at[pl.ds(off, CHUNK)], sv)
        pltpu.make_async_copy(rv, agg.at[dv], ssem).wait()
        pltpu.sync_copy(dst_hbm.at[pl.ds(off, CHUNK)], dv)
        pltpu.async_copy(h_hbm.at[sv], rv, gsem).wait()
        pltpu.async_copy(rv, agg.at[dv], ssem, add=True)

    def body(g, carry):
        for b, (sv, dv, rv, ssem) in enumerate(bufs):
            step(g * 2 + b, sv, dv, rv, ssem)
        return carry

    lax.fori_loop(0, CHUNKS // 2, body, 0)
    step(CHUNKS - 1, *bufs[0])
    for (sv, dv, rv, ssem) in bufs:
        pltpu.make_async_copy(rv, agg.at[dv], ssem).wait()
    plsc.subcore_barrier()

    pltpu.sync_copy(agg.at[pl.ds(row0, ROWS_PER_SUB)],
                    out_hbm.at[c, pl.ds(row0, ROWS_PER_SUB)])


@jax.jit
def _sc_scatter(h, src_pad, dst_pad, zeros_blk):
    mesh = plsc.VectorSubcoreMesh(core_axis_name="c", subcore_axis_name="s")
    f = pl.kernel(
        _sc_scatter_body,
        out_type=jax.ShapeDtypeStruct((NC, N_PAD, D), jnp.float32),
        mesh=mesh,
        scratch_types=[
            pltpu.VMEM((CHUNK,), jnp.int32),
            pltpu.VMEM((CHUNK,), jnp.int32),
            pltpu.VMEM((CHUNK,), jnp.int32),
            pltpu.VMEM((CHUNK,), jnp.int32),
            pltpu.VMEM((CHUNK, D), jnp.float32),
            pltpu.VMEM((CHUNK, D), jnp.float32),
            pltpu.VMEM_SHARED((N_PAD, D), jnp.float32),
            pltpu.SemaphoreType.DMA,
            pltpu.SemaphoreType.DMA,
            pltpu.SemaphoreType.DMA,
        ],
    )
    return f(h, src_pad, dst_pad, zeros_blk)


_EPS = 1e-5
_PREC = jax.lax.Precision.HIGHEST


def _bn_relu(y, g, b):
    m = jnp.mean(y, axis=0, keepdims=True)
    v = jnp.mean((y - m) ** 2, axis=0, keepdims=True)
    return jnp.maximum(g * (y - m) * lax.rsqrt(v + _EPS) + b, 0.0)


def _mlp(h, w1, b1, g1, be1, w2, b2, g2, be2):
    y = jnp.dot(h, w1, preferred_element_type=jnp.float32, precision=_PREC) + b1
    y = _bn_relu(y, g1, be1)
    y = jnp.dot(y, w2, preferred_element_type=jnp.float32, precision=_PREC) + b2
    return _bn_relu(y, g2, be2)


def _mlp1_body(x_ref, p0_ref, p1_ref,
               w1_ref, b1_ref, g1_ref, be1_ref,
               w2_ref, b2_ref, g2_ref, be2_ref, out_ref):
    h = x_ref[...] + p0_ref[:N] + p1_ref[:N]
    out_ref[...] = _mlp(h, w1_ref[...], b1_ref[...], g1_ref[...], be1_ref[...],
                        w2_ref[...], b2_ref[...], g2_ref[...], be2_ref[...])


def _mlp2_body(h1_ref, p0_ref, p1_ref,
               w1_ref, b1_ref, g1_ref, be1_ref,
               w2_ref, b2_ref, g2_ref, be2_ref,
               wa_ref, wb_ref, lb_ref, out_ref):
    h1 = h1_ref[...]
    h = h1 + p0_ref[:N] + p1_ref[:N]
    h2 = _mlp(h, w1_ref[...], b1_ref[...], g1_ref[...], be1_ref[...],
              w2_ref[...], b2_ref[...], g2_ref[...], be2_ref[...])
    out_ref[...] = (
        jnp.dot(h1, wa_ref[...], preferred_element_type=jnp.float32,
                precision=_PREC)
        + jnp.dot(h2, wb_ref[...], preferred_element_type=jnp.float32,
                  precision=_PREC)
        + lb_ref[...])


def _vmem_specs(n):
    return [pl.BlockSpec(memory_space=pltpu.VMEM) for _ in range(n)]


def _mlp1(x, parts, p):
    return pl.pallas_call(
        _mlp1_body,
        out_shape=jax.ShapeDtypeStruct((N, D), jnp.float32),
        in_specs=_vmem_specs(11),
        out_specs=pl.BlockSpec(memory_space=pltpu.VMEM),
    )(x, parts[0], parts[1],
      p["w1"], p["b1"].reshape(1, D), p["g1"].reshape(1, D),
      p["be1"].reshape(1, D),
      p["w2"], p["b2"].reshape(1, D), p["g2"].reshape(1, D),
      p["be2"].reshape(1, D))


def _mlp2(h1, parts, p, lin_w, lin_b):
    return pl.pallas_call(
        _mlp2_body,
        out_shape=jax.ShapeDtypeStruct((N, D), jnp.float32),
        in_specs=_vmem_specs(14),
        out_specs=pl.BlockSpec(memory_space=pltpu.VMEM),
    )(h1, parts[0], parts[1],
      p["w1"], p["b1"].reshape(1, D), p["g1"].reshape(1, D),
      p["be1"].reshape(1, D),
      p["w2"], p["b2"].reshape(1, D), p["g2"].reshape(1, D),
      p["be2"].reshape(1, D),
      lin_w[:D], lin_w[D:], lin_b.reshape(1, D))


def kernel(x, edge_index, params):
    src = edge_index[0]
    dst = edge_index[1]
    pad = E_PAD - E
    src_pad = jnp.concatenate([src, jnp.zeros((pad,), jnp.int32)])
    dst_pad = jnp.concatenate([dst, jnp.full((pad,), N, jnp.int32)])
    zeros_blk = jnp.zeros((ROWS_PER_SUB, D), jnp.float32)

    parts1 = _sc_scatter(x, src_pad, dst_pad, zeros_blk)
    h1 = _mlp1(x, parts1, params["conv1"])
    parts2 = _sc_scatter(h1, src_pad, dst_pad, zeros_blk)
    return _mlp2(h1, parts2, params["conv2"], params["lin_w"], params["lin_b"])


# speedup vs baseline: 2.2055x; 1.0578x over previous
import functools

import jax
import jax.numpy as jnp
from jax import lax
from jax.experimental import pallas as pl
from jax.experimental.pallas import tpu as pltpu
from jax.experimental.pallas import tpu_sc as plsc

N = 10000
E = 320000
D = 128

NC = 2
NS = 16
NW = NC * NS
CHUNK = 128
CHUNKS = 79
EPW = CHUNKS * CHUNK
E_PAD = NW * EPW
N_PAD = 10112
ROWS_PER_SUB = N_PAD // NS


def _sc_scatter_body(h_hbm, src_hbm, dst_hbm, zeros_hbm, out_hbm,
                     src_v0, src_v1, dst_v0, dst_v1, rows_v0, rows_v1,
                     agg, gsem, ssem0, ssem1):
    c = lax.axis_index("c")
    s = lax.axis_index("s")
    wid = s * NC + c

    row0 = s * ROWS_PER_SUB
    pltpu.sync_copy(zeros_hbm, agg.at[pl.ds(row0, ROWS_PER_SUB)])
    plsc.subcore_barrier()

    ebase = wid * EPW
    bufs = ((src_v0, dst_v0, rows_v0, ssem0),
            (src_v1, dst_v1, rows_v1, ssem1))

    dummy = E_PAD - CHUNK
    for (sv, dv, rv, ssem) in bufs:
        pltpu.sync_copy(dst_hbm.at[pl.ds(dummy, CHUNK)], dv)
        pltpu.sync_copy(zeros_hbm.at[pl.ds(0, CHUNK)], rv)
        pltpu.async_copy(rv, agg.at[dv], ssem, add=True)

    def step(k, sv, dv, rv, ssem):
        off = ebase + k * CHUNK
        pltpu.sync_copy(src_hbm.at[pl.ds(off, CHUNK)], sv)
        pltpu.make_async_copy(rv, agg.at[dv], ssem).wait()
        pltpu.sync_copy(dst_hbm.at[pl.ds(off, CHUNK)], dv)
        pltpu.async_copy(h_hbm.at[sv], rv, gsem).wait()
        pltpu.async_copy(rv, agg.at[dv], ssem, add=True)

    def body(g, carry):
        for b, (sv, dv, rv, ssem) in enumerate(bufs):
            step(g * 2 + b, sv, dv, rv, ssem)
        return carry

    lax.fori_loop(0, CHUNKS // 2, body, 0)
    step(CHUNKS - 1, *bufs[0])
    for (sv, dv, rv, ssem) in bufs:
        pltpu.make_async_copy(rv, agg.at[dv], ssem).wait()
    plsc.subcore_barrier()

    pltpu.sync_copy(agg.at[pl.ds(row0, ROWS_PER_SUB)],
                    out_hbm.at[c, pl.ds(row0, ROWS_PER_SUB)])


@jax.jit
def _sc_scatter(h, src_pad, dst_pad, zeros_blk):
    mesh = plsc.VectorSubcoreMesh(core_axis_name="c", subcore_axis_name="s")
    f = pl.kernel(
        _sc_scatter_body,
        out_type=jax.ShapeDtypeStruct((NC, N_PAD, D), jnp.float32),
        mesh=mesh,
        scratch_types=[
            pltpu.VMEM((CHUNK,), jnp.int32),
            pltpu.VMEM((CHUNK,), jnp.int32),
            pltpu.VMEM((CHUNK,), jnp.int32),
            pltpu.VMEM((CHUNK,), jnp.int32),
            pltpu.VMEM((CHUNK, D), jnp.float32),
            pltpu.VMEM((CHUNK, D), jnp.float32),
            pltpu.VMEM_SHARED((N_PAD, D), jnp.float32),
            pltpu.SemaphoreType.DMA,
            pltpu.SemaphoreType.DMA,
            pltpu.SemaphoreType.DMA,
        ],
    )
    return f(h, src_pad, dst_pad, zeros_blk)


_EPS = 1e-5
_PREC = None


def _bn_relu(y, g, b):
    m = jnp.mean(y, axis=0, keepdims=True)
    v = jnp.mean((y - m) ** 2, axis=0, keepdims=True)
    return jnp.maximum(g * (y - m) * lax.rsqrt(v + _EPS) + b, 0.0)


def _mlp(h, w1, b1, g1, be1, w2, b2, g2, be2):
    y = jnp.dot(h, w1, preferred_element_type=jnp.float32, precision=_PREC) + b1
    y = _bn_relu(y, g1, be1)
    y = jnp.dot(y, w2, preferred_element_type=jnp.float32, precision=_PREC) + b2
    return _bn_relu(y, g2, be2)


def _mlp1_body(x_ref, p0_ref, p1_ref,
               w1_ref, b1_ref, g1_ref, be1_ref,
               w2_ref, b2_ref, g2_ref, be2_ref, out_ref):
    h = x_ref[...] + p0_ref[:N] + p1_ref[:N]
    out_ref[...] = _mlp(h, w1_ref[...], b1_ref[...], g1_ref[...], be1_ref[...],
                        w2_ref[...], b2_ref[...], g2_ref[...], be2_ref[...])


def _mlp2_body(h1_ref, p0_ref, p1_ref,
               w1_ref, b1_ref, g1_ref, be1_ref,
               w2_ref, b2_ref, g2_ref, be2_ref,
               wa_ref, wb_ref, lb_ref, out_ref):
    h1 = h1_ref[...]
    h = h1 + p0_ref[:N] + p1_ref[:N]
    h2 = _mlp(h, w1_ref[...], b1_ref[...], g1_ref[...], be1_ref[...],
              w2_ref[...], b2_ref[...], g2_ref[...], be2_ref[...])
    out_ref[...] = (
        jnp.dot(h1, wa_ref[...], preferred_element_type=jnp.float32,
                precision=_PREC)
        + jnp.dot(h2, wb_ref[...], preferred_element_type=jnp.float32,
                  precision=_PREC)
        + lb_ref[...])


def _vmem_specs(n):
    return [pl.BlockSpec(memory_space=pltpu.VMEM) for _ in range(n)]


def _mlp1(x, parts, p):
    return pl.pallas_call(
        _mlp1_body,
        out_shape=jax.ShapeDtypeStruct((N, D), jnp.float32),
        in_specs=_vmem_specs(11),
        out_specs=pl.BlockSpec(memory_space=pltpu.VMEM),
    )(x, parts[0], parts[1],
      p["w1"], p["b1"].reshape(1, D), p["g1"].reshape(1, D),
      p["be1"].reshape(1, D),
      p["w2"], p["b2"].reshape(1, D), p["g2"].reshape(1, D),
      p["be2"].reshape(1, D))


def _mlp2(h1, parts, p, lin_w, lin_b):
    return pl.pallas_call(
        _mlp2_body,
        out_shape=jax.ShapeDtypeStruct((N, D), jnp.float32),
        in_specs=_vmem_specs(14),
        out_specs=pl.BlockSpec(memory_space=pltpu.VMEM),
    )(h1, parts[0], parts[1],
      p["w1"], p["b1"].reshape(1, D), p["g1"].reshape(1, D),
      p["be1"].reshape(1, D),
      p["w2"], p["b2"].reshape(1, D), p["g2"].reshape(1, D),
      p["be2"].reshape(1, D),
      lin_w[:D], lin_w[D:], lin_b.reshape(1, D))


def kernel(x, edge_index, params):
    src = edge_index[0]
    dst = edge_index[1]
    pad = E_PAD - E
    src_pad = jnp.concatenate([src, jnp.zeros((pad,), jnp.int32)])
    dst_pad = jnp.concatenate([dst, jnp.full((pad,), N, jnp.int32)])
    zeros_blk = jnp.zeros((ROWS_PER_SUB, D), jnp.float32)

    parts1 = _sc_scatter(x, src_pad, dst_pad, zeros_blk)
    h1 = _mlp1(x, parts1, params["conv1"])
    parts2 = _sc_scatter(h1, src_pad, dst_pad, zeros_blk)
    return _mlp2(h1, parts2, params["conv2"], params["lin_w"], params["lin_b"])


# speedup vs baseline: 2.3938x; 1.0854x over previous
import functools

import jax
import jax.numpy as jnp
from jax import lax
from jax.experimental import pallas as pl
from jax.experimental.pallas import tpu as pltpu
from jax.experimental.pallas import tpu_sc as plsc

N = 10000
E = 320000
D = 128

NC = 2
NS = 16
NW = NC * NS
CHUNK = 128
CHUNKS = 79
EPW = CHUNKS * CHUNK
E_PAD = NW * EPW
N_PAD = 10112
ROWS_PER_SUB = N_PAD // NS


def _sc_scatter_body(h_hbm, src_hbm, dst_hbm, zeros_hbm, out_hbm,
                     src_v0, src_v1, dst_v0, dst_v1, rows_v0, rows_v1,
                     agg, gsem, ssem0, ssem1, isem0, isem1):
    c = lax.axis_index("c")
    s = lax.axis_index("s")
    wid = s * NC + c

    row0 = s * ROWS_PER_SUB
    pltpu.sync_copy(zeros_hbm, agg.at[pl.ds(row0, ROWS_PER_SUB)])
    plsc.subcore_barrier()

    ebase = wid * EPW
    bufs = ((src_v0, dst_v0, rows_v0, ssem0, isem0),
            (src_v1, dst_v1, rows_v1, ssem1, isem1))

    dummy = E_PAD - CHUNK
    for b, (sv, dv, rv, ssem, isem) in enumerate(bufs):
        pltpu.sync_copy(dst_hbm.at[pl.ds(dummy, CHUNK)], dv)
        pltpu.sync_copy(zeros_hbm.at[pl.ds(0, CHUNK)], rv)
        pltpu.async_copy(rv, agg.at[dv], ssem, add=True)
        pltpu.async_copy(src_hbm.at[pl.ds(ebase + b * CHUNK, CHUNK)], sv,
                         isem)

    def step(k, sv, dv, rv, ssem, isem):
        off = ebase + k * CHUNK
        pltpu.make_async_copy(src_hbm.at[pl.ds(off, CHUNK)], sv,
                              isem).wait()
        pltpu.make_async_copy(rv, agg.at[dv], ssem).wait()
        pltpu.sync_copy(dst_hbm.at[pl.ds(off, CHUNK)], dv)
        pltpu.async_copy(h_hbm.at[sv], rv, gsem).wait()
        pltpu.async_copy(rv, agg.at[dv], ssem, add=True)

        @pl.when(k + 2 < CHUNKS)
        def _():
            pltpu.async_copy(
                src_hbm.at[pl.ds(off + 2 * CHUNK, CHUNK)], sv, isem)

    def body(g, carry):
        for b, buf in enumerate(bufs):
            step(g * 2 + b, *buf)
        return carry

    lax.fori_loop(0, CHUNKS // 2, body, 0)
    step(CHUNKS - 1, *bufs[0])
    for (sv, dv, rv, ssem, isem) in bufs:
        pltpu.make_async_copy(rv, agg.at[dv], ssem).wait()
    plsc.subcore_barrier()

    pltpu.sync_copy(agg.at[pl.ds(row0, ROWS_PER_SUB)],
                    out_hbm.at[c, pl.ds(row0, ROWS_PER_SUB)])


@jax.jit
def _sc_scatter(h, src_pad, dst_pad, zeros_blk):
    mesh = plsc.VectorSubcoreMesh(core_axis_name="c", subcore_axis_name="s")
    f = pl.kernel(
        _sc_scatter_body,
        out_type=jax.ShapeDtypeStruct((NC, N_PAD, D), jnp.float32),
        mesh=mesh,
        scratch_types=[
            pltpu.VMEM((CHUNK,), jnp.int32),
            pltpu.VMEM((CHUNK,), jnp.int32),
            pltpu.VMEM((CHUNK,), jnp.int32),
            pltpu.VMEM((CHUNK,), jnp.int32),
            pltpu.VMEM((CHUNK, D), jnp.float32),
            pltpu.VMEM((CHUNK, D), jnp.float32),
            pltpu.VMEM_SHARED((N_PAD, D), jnp.float32),
            pltpu.SemaphoreType.DMA,
            pltpu.SemaphoreType.DMA,
            pltpu.SemaphoreType.DMA,
            pltpu.SemaphoreType.DMA,
            pltpu.SemaphoreType.DMA,
        ],
    )
    return f(h, src_pad, dst_pad, zeros_blk)


_EPS = 1e-5
_PREC = None


def _bn_relu(y, g, b):
    m = jnp.mean(y, axis=0, keepdims=True)
    v = jnp.mean((y - m) ** 2, axis=0, keepdims=True)
    return jnp.maximum(g * (y - m) * lax.rsqrt(v + _EPS) + b, 0.0)


def _mlp(h, w1, b1, g1, be1, w2, b2, g2, be2):
    y = jnp.dot(h, w1, preferred_element_type=jnp.float32, precision=_PREC) + b1
    y = _bn_relu(y, g1, be1)
    y = jnp.dot(y, w2, preferred_element_type=jnp.float32, precision=_PREC) + b2
    return _bn_relu(y, g2, be2)


def _mlp1_body(x_ref, p0_ref, p1_ref,
               w1_ref, b1_ref, g1_ref, be1_ref,
               w2_ref, b2_ref, g2_ref, be2_ref, out_ref):
    h = x_ref[...] + p0_ref[:N] + p1_ref[:N]
    out_ref[...] = _mlp(h, w1_ref[...], b1_ref[...], g1_ref[...], be1_ref[...],
                        w2_ref[...], b2_ref[...], g2_ref[...], be2_ref[...])


def _mlp2_body(h1_ref, p0_ref, p1_ref,
               w1_ref, b1_ref, g1_ref, be1_ref,
               w2_ref, b2_ref, g2_ref, be2_ref,
               wa_ref, wb_ref, lb_ref, out_ref):
    h1 = h1_ref[...]
    h = h1 + p0_ref[:N] + p1_ref[:N]
    h2 = _mlp(h, w1_ref[...], b1_ref[...], g1_ref[...], be1_ref[...],
              w2_ref[...], b2_ref[...], g2_ref[...], be2_ref[...])
    out_ref[...] = (
        jnp.dot(h1, wa_ref[...], preferred_element_type=jnp.float32,
                precision=_PREC)
        + jnp.dot(h2, wb_ref[...], preferred_element_type=jnp.float32,
                  precision=_PREC)
        + lb_ref[...])


def _vmem_specs(n):
    return [pl.BlockSpec(memory_space=pltpu.VMEM) for _ in range(n)]


def _mlp1(x, parts, p):
    return pl.pallas_call(
        _mlp1_body,
        out_shape=jax.ShapeDtypeStruct((N, D), jnp.float32),
        in_specs=_vmem_specs(11),
        out_specs=pl.BlockSpec(memory_space=pltpu.VMEM),
    )(x, parts[0], parts[1],
      p["w1"], p["b1"].reshape(1, D), p["g1"].reshape(1, D),
      p["be1"].reshape(1, D),
      p["w2"], p["b2"].reshape(1, D), p["g2"].reshape(1, D),
      p["be2"].reshape(1, D))


def _mlp2(h1, parts, p, lin_w, lin_b):
    return pl.pallas_call(
        _mlp2_body,
        out_shape=jax.ShapeDtypeStruct((N, D), jnp.float32),
        in_specs=_vmem_specs(14),
        out_specs=pl.BlockSpec(memory_space=pltpu.VMEM),
    )(h1, parts[0], parts[1],
      p["w1"], p["b1"].reshape(1, D), p["g1"].reshape(1, D),
      p["be1"].reshape(1, D),
      p["w2"], p["b2"].reshape(1, D), p["g2"].reshape(1, D),
      p["be2"].reshape(1, D),
      lin_w[:D], lin_w[D:], lin_b.reshape(1, D))


def kernel(x, edge_index, params):
    src = edge_index[0]
    dst = edge_index[1]
    pad = E_PAD - E
    src_pad = jnp.concatenate([src, jnp.zeros((pad,), jnp.int32)])
    dst_pad = jnp.concatenate([dst, jnp.full((pad,), N, jnp.int32)])
    zeros_blk = jnp.zeros((ROWS_PER_SUB, D), jnp.float32)

    parts1 = _sc_scatter(x, src_pad, dst_pad, zeros_blk)
    h1 = _mlp1(x, parts1, params["conv1"])
    parts2 = _sc_scatter(h1, src_pad, dst_pad, zeros_blk)
    return _mlp2(h1, parts2, params["conv2"], params["lin_w"], params["lin_b"])


# speedup vs baseline: 2.3967x; 1.0012x over previous
import jax
import jax.numpy as jnp
from jax import lax
from jax.experimental import pallas as pl
from jax.experimental.pallas import tpu as pltpu
from jax.experimental.pallas import tpu_sc as plsc

N = 10000
E = 320000
D = 128

NC = 2
NS = 16
NW = NC * NS
CHUNK = 128
CHUNKS = 79
EPW = CHUNKS * CHUNK
E_PAD = NW * EPW
N_PAD = 10112
ROWS_PER_SUB = N_PAD // NS


def _sc_scatter_body(h_hbm, src_hbm, dst_hbm, zeros_hbm, out_hbm,
                     src_v0, src_v1, dst_v0, dst_v1, rows_v0, rows_v1,
                     agg, gsem, ssem0, ssem1, isem0, isem1):
    c = lax.axis_index("c")
    s = lax.axis_index("s")
    wid = s * NC + c

    row0 = s * ROWS_PER_SUB
    pltpu.sync_copy(zeros_hbm, agg.at[pl.ds(row0, ROWS_PER_SUB)])
    plsc.subcore_barrier()

    ebase = wid * EPW
    bufs = ((src_v0, dst_v0, rows_v0, ssem0, isem0),
            (src_v1, dst_v1, rows_v1, ssem1, isem1))

    dummy = E_PAD - CHUNK
    for b, (sv, dv, rv, ssem, isem) in enumerate(bufs):
        pltpu.sync_copy(dst_hbm.at[pl.ds(dummy, CHUNK)], dv)
        pltpu.sync_copy(zeros_hbm.at[pl.ds(0, CHUNK)], rv)
        pltpu.async_copy(rv, agg.at[dv], ssem, add=True)
        pltpu.async_copy(src_hbm.at[pl.ds(ebase + b * CHUNK, CHUNK)], sv,
                         isem)

    def step(k, sv, dv, rv, ssem, isem):
        off = ebase + k * CHUNK
        pltpu.make_async_copy(src_hbm.at[pl.ds(off, CHUNK)], sv,
                              isem).wait()
        pltpu.make_async_copy(rv, agg.at[dv], ssem).wait()
        pltpu.sync_copy(dst_hbm.at[pl.ds(off, CHUNK)], dv)
        pltpu.async_copy(h_hbm.at[sv], rv, gsem).wait()
        pltpu.async_copy(rv, agg.at[dv], ssem, add=True)

        @pl.when(k + 2 < CHUNKS)
        def _():
            pltpu.async_copy(
                src_hbm.at[pl.ds(off + 2 * CHUNK, CHUNK)], sv, isem)

    def body(g, carry):
        for b, buf in enumerate(bufs):
            step(g * 2 + b, *buf)
        return carry

    lax.fori_loop(0, CHUNKS // 2, body, 0)
    step(CHUNKS - 1, *bufs[0])
    for (sv, dv, rv, ssem, isem) in bufs:
        pltpu.make_async_copy(rv, agg.at[dv], ssem).wait()
    plsc.subcore_barrier()

    pltpu.sync_copy(agg.at[pl.ds(row0, ROWS_PER_SUB)],
                    out_hbm.at[c, pl.ds(row0, ROWS_PER_SUB)])


@jax.jit
def _sc_scatter(h, src_pad, dst_pad, zeros_blk):
    mesh = plsc.VectorSubcoreMesh(core_axis_name="c", subcore_axis_name="s")
    f = pl.kernel(
        _sc_scatter_body,
        out_type=jax.ShapeDtypeStruct((NC, N_PAD, D), jnp.float32),
        mesh=mesh,
        scratch_types=[
            pltpu.VMEM((CHUNK,), jnp.int32),
            pltpu.VMEM((CHUNK,), jnp.int32),
            pltpu.VMEM((CHUNK,), jnp.int32),
            pltpu.VMEM((CHUNK,), jnp.int32),
            pltpu.VMEM((CHUNK, D), jnp.float32),
            pltpu.VMEM((CHUNK, D), jnp.float32),
            pltpu.VMEM_SHARED((N_PAD, D), jnp.float32),
            pltpu.SemaphoreType.DMA,
            pltpu.SemaphoreType.DMA,
            pltpu.SemaphoreType.DMA,
            pltpu.SemaphoreType.DMA,
            pltpu.SemaphoreType.DMA,
        ],
    )
    return f(h, src_pad, dst_pad, zeros_blk)


_EPS = 1e-5
_PREC = None


def _bn_relu(y, g, b):
    m = jnp.mean(y, axis=0, keepdims=True)
    v = jnp.mean((y - m) ** 2, axis=0, keepdims=True)
    return jnp.maximum(g * (y - m) * lax.rsqrt(v + _EPS) + b, 0.0)


def _mlp(h, w1, b1, g1, be1, w2, b2, g2, be2):
    y = jnp.dot(h, w1, preferred_element_type=jnp.float32, precision=_PREC) + b1
    y = _bn_relu(y, g1, be1)
    y = jnp.dot(y, w2, preferred_element_type=jnp.float32, precision=_PREC) + b2
    return _bn_relu(y, g2, be2)


def _mlp1_body(x_ref, p0_ref, p1_ref,
               w1_ref, b1_ref, g1_ref, be1_ref,
               w2_ref, b2_ref, g2_ref, be2_ref, out_ref):
    h = x_ref[...] + p0_ref[:N] + p1_ref[:N]
    out_ref[...] = _mlp(h, w1_ref[...], b1_ref[...], g1_ref[...], be1_ref[...],
                        w2_ref[...], b2_ref[...], g2_ref[...], be2_ref[...])


def _mlp2_body(h1_ref, p0_ref, p1_ref,
               w1_ref, b1_ref, g1_ref, be1_ref,
               w2_ref, b2_ref, g2_ref, be2_ref,
               wa_ref, wb_ref, lb_ref, out_ref):
    h1 = h1_ref[...]
    h = h1 + p0_ref[:N] + p1_ref[:N]
    h2 = _mlp(h, w1_ref[...], b1_ref[...], g1_ref[...], be1_ref[...],
              w2_ref[...], b2_ref[...], g2_ref[...], be2_ref[...])
    out_ref[...] = (
        jnp.dot(h1, wa_ref[...], preferred_element_type=jnp.float32,
                precision=_PREC)
        + jnp.dot(h2, wb_ref[...], preferred_element_type=jnp.float32,
                  precision=_PREC)
        + lb_ref[...])


def _vmem_specs(n):
    return [pl.BlockSpec(memory_space=pltpu.VMEM) for _ in range(n)]


def _mlp1(x, parts, p):
    return pl.pallas_call(
        _mlp1_body,
        out_shape=jax.ShapeDtypeStruct((N, D), jnp.float32),
        in_specs=_vmem_specs(11),
        out_specs=pl.BlockSpec(memory_space=pltpu.VMEM),
    )(x, parts[0], parts[1],
      p["w1"], p["b1"].reshape(1, D), p["g1"].reshape(1, D),
      p["be1"].reshape(1, D),
      p["w2"], p["b2"].reshape(1, D), p["g2"].reshape(1, D),
      p["be2"].reshape(1, D))


def _mlp2(h1, parts, p, lin_w, lin_b):
    return pl.pallas_call(
        _mlp2_body,
        out_shape=jax.ShapeDtypeStruct((N, D), jnp.float32),
        in_specs=_vmem_specs(14),
        out_specs=pl.BlockSpec(memory_space=pltpu.VMEM),
    )(h1, parts[0], parts[1],
      p["w1"], p["b1"].reshape(1, D), p["g1"].reshape(1, D),
      p["be1"].reshape(1, D),
      p["w2"], p["b2"].reshape(1, D), p["g2"].reshape(1, D),
      p["be2"].reshape(1, D),
      lin_w[:D], lin_w[D:], lin_b.reshape(1, D))


def kernel(x, edge_index, params):
    src = edge_index[0]
    dst = edge_index[1]
    pad = E_PAD - E
    src_pad = jnp.concatenate([src, jnp.zeros((pad,), jnp.int32)])
    dst_pad = jnp.concatenate([dst, jnp.full((pad,), N, jnp.int32)])
    zeros_blk = jnp.zeros((ROWS_PER_SUB, D), jnp.float32)

    parts1 = _sc_scatter(x, src_pad, dst_pad, zeros_blk)
    h1 = _mlp1(x, parts1, params["conv1"])
    parts2 = _sc_scatter(h1, src_pad, dst_pad, zeros_blk)
    return _mlp2(h1, parts2, params["conv2"], params["lin_w"], params["lin_b"])
